# baseline (speedup 1.0000x reference)
import jax
import jax.numpy as jnp
from jax import lax
from jax.experimental import pallas as pl
from jax.experimental.pallas import tpu as pltpu

_BLK = 128
_K = 32
_LOG2E = 1.4426950408889634


def _top32_desc(s):
    n, b = s.shape
    lists = n // _K
    a = s.reshape(_K, lists, b)
    for sz in (2, 4, 8, 16, 32):
        st = sz // 2
        while st >= 1:
            g = _K // (2 * st)
            a4 = a.reshape(g, 2, st, lists, b)
            x, y = a4[:, 0], a4[:, 1]
            mx = jnp.maximum(x, y)
            mn = jnp.minimum(x, y)
            if sz == _K:
                first, second = mx, mn
            else:
                q = lax.broadcasted_iota(jnp.int32, (g, 1, 1, 1), 0)
                dm = ((q * (2 * st)) // sz) % 2 == 0
                first = jnp.where(dm, mx, mn)
                second = jnp.where(dm, mn, mx)
            a = jnp.stack([first, second], axis=1).reshape(_K, lists, b)
            st //= 2
    cur = lists
    while cur > 1:
        h = cur // 2
        x = a[:, :h]
        y = a[:, h:]
        yr = jnp.stack([y[_K - 1 - k] for k in range(_K)], axis=0)
        a = jnp.maximum(x, yr)
        for st in (16, 8, 4, 2, 1):
            g = _K // (2 * st)
            a4 = a.reshape(g, 2, st, h, b)
            x2, y2 = a4[:, 0], a4[:, 1]
            a = jnp.stack([jnp.maximum(x2, y2), jnp.minimum(x2, y2)],
                          axis=1).reshape(_K, h, b)
        cur = h
    return a.reshape(_K, b)


def _stats_kernel(x_ref, w_ref, c_ref, tb_ref, ab_ref, c_vmem, s_buf):
    j = pl.program_id(0)
    nb = x_ref.shape[0] // _BLK

    @pl.when(j == 0)
    def _():
        c = jax.nn.relu(
            lax.dot_general(x_ref[...], w_ref[...], (((1,), (1,)), ((), ())),
                            preferred_element_type=jnp.float32))
        c_vmem[...] = c
        c_ref[...] = c

    jm = jnp.minimum(j, nb - 1)
    c_blk = c_vmem[pl.ds(jm * _BLK, _BLK), :]
    s_buf[j % 2] = lax.dot_general(
        c_vmem[...], c_blk, (((1,), (1,)), ((), ())),
        preferred_element_type=jnp.float32)
    s = s_buf[(j + 1) % 2]
    vals = _top32_desc(s)
    big_m = vals[0:1, :]
    t = vals[_K - 1:_K, :]
    z = jnp.sum(jnp.exp(vals - big_m), axis=0, keepdims=True)
    coef = big_m * _LOG2E + jnp.log2(z)
    tb_ref[...] = jnp.broadcast_to(t.reshape(_BLK, 1), (_BLK, _BLK))
    ab_ref[...] = jnp.broadcast_to(coef.reshape(_BLK, 1), (_BLK, _BLK))


def _adj_kernel(c_full_ref, c_blk_ref, tb_ref, ab_ref, o_ref):
    s = lax.dot_general(
        c_full_ref[...], c_blk_ref[...], (((1,), (1,)), ((), ())),
        preferred_element_type=jnp.float32)
    o_ref[...] = jnp.where(
        s >= tb_ref[...],
        jnp.exp2(s * _LOG2E - ab_ref[...]),
        0.0)


def _build(n, d, h, interpret=False):
    nb = n // _BLK

    stats = pl.pallas_call(
        _stats_kernel,
        grid=(nb + 1,),
        in_specs=[
            pl.BlockSpec((n, d), lambda j: (0, 0)),
            pl.BlockSpec((h, d), lambda j: (0, 0)),
        ],
        out_specs=[
            pl.BlockSpec((n, h), lambda j: (0, 0)),
            pl.BlockSpec((_BLK, _BLK), lambda j: (jnp.maximum(j - 1, 0), 0)),
            pl.BlockSpec((_BLK, _BLK), lambda j: (jnp.maximum(j - 1, 0), 0)),
        ],
        out_shape=[
            jax.ShapeDtypeStruct((n, h), jnp.float32),
            jax.ShapeDtypeStruct((n, _BLK), jnp.float32),
            jax.ShapeDtypeStruct((n, _BLK), jnp.float32),
        ],
        scratch_shapes=[pltpu.VMEM((n, h), jnp.float32),
                        pltpu.VMEM((2, n, _BLK), jnp.float32)],
        compiler_params=pltpu.CompilerParams(
            dimension_semantics=("arbitrary",)),
        interpret=interpret,
    )

    adj_call = pl.pallas_call(
        _adj_kernel,
        grid=(nb,),
        in_specs=[
            pl.BlockSpec((n, h), lambda j: (0, 0)),
            pl.BlockSpec((_BLK, h), lambda j: (j, 0)),
            pl.BlockSpec((n, _BLK), lambda j: (0, 0)),
            pl.BlockSpec((n, _BLK), lambda j: (0, 0)),
        ],
        out_specs=pl.BlockSpec((n, _BLK), lambda j: (0, j)),
        out_shape=jax.ShapeDtypeStruct((n, n), jnp.float32),
        compiler_params=pltpu.CompilerParams(
            dimension_semantics=("arbitrary",)),
        interpret=interpret,
    )
    return stats, adj_call


def _run(node_features, W, interpret=False):
    n, d = node_features.shape
    h = W.shape[0]
    stats, adj_call = _build(n, d, h, interpret)
    c, tb, ab = stats(node_features, W)
    adj = adj_call(c, c, tb, ab)
    return (node_features, adj)


def kernel(node_features, W):
    return _run(node_features, W, interpret=False)


# speedup vs baseline: 1.0872x; 1.0872x over previous
import jax
import jax.numpy as jnp
from jax import lax
from jax.experimental import pallas as pl
from jax.experimental.pallas import tpu as pltpu

_BLK = 128
_K = 32
_LOG2E = 1.4426950408889634


def _top32_desc(s):
    n, b = s.shape
    lists = n // _K
    a = s.reshape(_K, lists, b)
    for sz in (2, 4, 8, 16, 32):
        st = sz // 2
        while st >= 1:
            g = _K // (2 * st)
            a4 = a.reshape(g, 2, st, lists, b)
            x, y = a4[:, 0], a4[:, 1]
            mx = jnp.maximum(x, y)
            mn = jnp.minimum(x, y)
            if sz == _K:
                first, second = mx, mn
            else:
                q = lax.broadcasted_iota(jnp.int32, (g, 1, 1, 1), 0)
                dm = ((q * (2 * st)) // sz) % 2 == 0
                first = jnp.where(dm, mx, mn)
                second = jnp.where(dm, mn, mx)
            a = jnp.stack([first, second], axis=1).reshape(_K, lists, b)
            st //= 2
    cur = lists
    while cur > 1:
        h = cur // 2
        x = a[:, :h]
        y = a[:, h:]
        yr = jnp.stack([y[_K - 1 - k] for k in range(_K)], axis=0)
        a = jnp.maximum(x, yr)
        for st in (16, 8, 4, 2, 1):
            g = _K // (2 * st)
            a4 = a.reshape(g, 2, st, h, b)
            x2, y2 = a4[:, 0], a4[:, 1]
            a = jnp.stack([jnp.maximum(x2, y2), jnp.minimum(x2, y2)],
                          axis=1).reshape(_K, h, b)
        cur = h
    return a.reshape(_K, b)


def _stats_kernel(x_ref, w_ref, c_ref, t_ref, a_ref, c_vmem):
    j = pl.program_id(0)

    @pl.when(j == 0)
    def _():
        c = jax.nn.relu(
            lax.dot_general(x_ref[...], w_ref[...], (((1,), (1,)), ((), ())),
                            preferred_element_type=jnp.float32))
        c_vmem[...] = c
        c_ref[...] = c

    c_blk = c_vmem[pl.ds(j * _BLK, _BLK), :]
    s = lax.dot_general(
        c_vmem[...], c_blk, (((1,), (1,)), ((), ())),
        preferred_element_type=jnp.float32)
    vals = _top32_desc(s)
    big_m = vals[0:1, :]
    t = vals[_K - 1:_K, :]
    z = jnp.sum(jnp.exp(vals - big_m), axis=0, keepdims=True)
    coef = big_m * _LOG2E + jnp.log2(z)
    t_ref[...] = t.reshape(1, 1, _BLK)
    a_ref[...] = coef.reshape(1, 1, _BLK)


def _adj_kernel(c_full_ref, c_blk_ref, t_ref, a_ref, o_ref):
    s = lax.dot_general(
        c_full_ref[...], c_blk_ref[...], (((1,), (1,)), ((), ())),
        preferred_element_type=jnp.float32)
    t = t_ref[0]
    a = a_ref[0]
    out = jnp.where(s >= t, jnp.exp2(s * _LOG2E - a), 0.0)
    o_ref[...] = out.T


def _build(n, d, h, interpret=False):
    nb = n // _BLK

    stats = pl.pallas_call(
        _stats_kernel,
        grid=(nb,),
        in_specs=[
            pl.BlockSpec((n, d), lambda j: (0, 0)),
            pl.BlockSpec((h, d), lambda j: (0, 0)),
        ],
        out_specs=[
            pl.BlockSpec((n, h), lambda j: (0, 0)),
            pl.BlockSpec((1, 1, _BLK), lambda j: (j, 0, 0)),
            pl.BlockSpec((1, 1, _BLK), lambda j: (j, 0, 0)),
        ],
        out_shape=[
            jax.ShapeDtypeStruct((n, h), jnp.float32),
            jax.ShapeDtypeStruct((nb, 1, _BLK), jnp.float32),
            jax.ShapeDtypeStruct((nb, 1, _BLK), jnp.float32),
        ],
        scratch_shapes=[pltpu.VMEM((n, h), jnp.float32)],
        compiler_params=pltpu.CompilerParams(
            dimension_semantics=("arbitrary",)),
        interpret=interpret,
    )

    adj_call = pl.pallas_call(
        _adj_kernel,
        grid=(nb,),
        in_specs=[
            pl.BlockSpec((n, h), lambda j: (0, 0)),
            pl.BlockSpec((_BLK, h), lambda j: (j, 0)),
            pl.BlockSpec((1, 1, _BLK), lambda j: (j, 0, 0)),
            pl.BlockSpec((1, 1, _BLK), lambda j: (j, 0, 0)),
        ],
        out_specs=pl.BlockSpec((_BLK, n), lambda j: (j, 0)),
        out_shape=jax.ShapeDtypeStruct((n, n), jnp.float32),
        compiler_params=pltpu.CompilerParams(
            dimension_semantics=("arbitrary",)),
        interpret=interpret,
    )
    return stats, adj_call


def _run(node_features, W, interpret=False):
    n, d = node_features.shape
    h = W.shape[0]
    stats, adj_call = _build(n, d, h, interpret)
    c, t, a = stats(node_features, W)
    adj = adj_call(c, c, t, a)
    return (node_features, adj)


def kernel(node_features, W):
    return _run(node_features, W, interpret=False)


# speedup vs baseline: 1.1002x; 1.0120x over previous
import jax
import jax.numpy as jnp
from jax import lax
from jax.experimental import pallas as pl
from jax.experimental.pallas import tpu as pltpu

_BLK = 128
_K = 32
_LOG2E = 1.4426950408889634


def _top32_desc(s):
    n, b = s.shape
    lists = n // _K
    a = s.reshape(_K, lists, b)
    for sz in (2, 4, 8, 16, 32):
        st = sz // 2
        while st >= 1:
            g = _K // (2 * st)
            a4 = a.reshape(g, 2, st, lists, b)
            x, y = a4[:, 0], a4[:, 1]
            mx = jnp.maximum(x, y)
            mn = jnp.minimum(x, y)
            if sz == _K:
                first, second = mx, mn
            else:
                q = lax.broadcasted_iota(jnp.int32, (g, 1, 1, 1), 0)
                dm = ((q * (2 * st)) // sz) % 2 == 0
                first = jnp.where(dm, mx, mn)
                second = jnp.where(dm, mn, mx)
            a = jnp.stack([first, second], axis=1).reshape(_K, lists, b)
            st //= 2
    cur = lists
    while cur > 1:
        h = cur // 2
        x = a[:, :h]
        y = a[:, h:]
        yr = jnp.stack([y[_K - 1 - k] for k in range(_K)], axis=0)
        a = jnp.maximum(x, yr)
        for st in (16, 8, 4, 2, 1):
            g = _K // (2 * st)
            a4 = a.reshape(g, 2, st, h, b)
            x2, y2 = a4[:, 0], a4[:, 1]
            a = jnp.stack([jnp.maximum(x2, y2), jnp.minimum(x2, y2)],
                          axis=1).reshape(_K, h, b)
        cur = h
    return a.reshape(_K, b)


def _stats_kernel(x_ref, w_ref, c_ref, t_ref, a_ref, c_vmem):
    j = pl.program_id(0)

    @pl.when(j == 0)
    def _():
        c = jax.nn.relu(
            lax.dot_general(x_ref[...], w_ref[...], (((1,), (1,)), ((), ())),
                            preferred_element_type=jnp.float32))
        c_vmem[...] = c
        c_ref[...] = c

    c_blk = c_vmem[pl.ds(j * _BLK, _BLK), :]
    s = lax.dot_general(
        c_vmem[...], c_blk, (((1,), (1,)), ((), ())),
        preferred_element_type=jnp.float32)
    vals = _top32_desc(s)
    big_m = vals[0:1, :]
    t = vals[_K - 1:_K, :]
    z = jnp.sum(jnp.exp(vals - big_m), axis=0, keepdims=True)
    coef = big_m * _LOG2E + jnp.log2(z)
    t_ref[...] = jnp.broadcast_to(t.reshape(_BLK, 1), (_BLK, _BLK))
    a_ref[...] = jnp.broadcast_to(coef.reshape(_BLK, 1), (_BLK, _BLK))


def _adj_kernel(c_full_ref, c_blk_ref, tb_ref, ab_ref, o_ref):
    s = lax.dot_general(
        c_full_ref[...], c_blk_ref[...], (((1,), (1,)), ((), ())),
        preferred_element_type=jnp.float32)
    o_ref[...] = jnp.where(
        s >= tb_ref[...],
        jnp.exp2(s * _LOG2E - ab_ref[...]),
        0.0)


def _build(n, d, h, interpret=False):
    nb = n // _BLK

    stats = pl.pallas_call(
        _stats_kernel,
        grid=(nb,),
        in_specs=[
            pl.BlockSpec((n, d), lambda j: (0, 0)),
            pl.BlockSpec((h, d), lambda j: (0, 0)),
        ],
        out_specs=[
            pl.BlockSpec((n, h), lambda j: (0, 0)),
            pl.BlockSpec((_BLK, _BLK), lambda j: (j, 0)),
            pl.BlockSpec((_BLK, _BLK), lambda j: (j, 0)),
        ],
        out_shape=[
            jax.ShapeDtypeStruct((n, h), jnp.float32),
            jax.ShapeDtypeStruct((n, _BLK), jnp.float32),
            jax.ShapeDtypeStruct((n, _BLK), jnp.float32),
        ],
        scratch_shapes=[pltpu.VMEM((n, h), jnp.float32)],
        compiler_params=pltpu.CompilerParams(
            dimension_semantics=("arbitrary",)),
        interpret=interpret,
    )

    adj_call = pl.pallas_call(
        _adj_kernel,
        grid=(nb,),
        in_specs=[
            pl.BlockSpec((n, h), lambda j: (0, 0)),
            pl.BlockSpec((_BLK, h), lambda j: (j, 0)),
            pl.BlockSpec((n, _BLK), lambda j: (0, 0)),
            pl.BlockSpec((n, _BLK), lambda j: (0, 0)),
        ],
        out_specs=pl.BlockSpec((n, _BLK), lambda j: (0, j)),
        out_shape=jax.ShapeDtypeStruct((n, n), jnp.float32),
        compiler_params=pltpu.CompilerParams(
            dimension_semantics=("parallel",)),
        interpret=interpret,
    )
    return stats, adj_call


def _run(node_features, W, interpret=False):
    n, d = node_features.shape
    h = W.shape[0]
    stats, adj_call = _build(n, d, h, interpret)
    c, tb, ab = stats(node_features, W)
    adj = adj_call(c, c, tb, ab)
    return (node_features, adj)


def kernel(node_features, W):
    return _run(node_features, W, interpret=False)
